# Initial kernel scaffold; baseline (speedup 1.0000x reference)
import jax
import jax.numpy as jnp
from jax.experimental import pallas as pl


def kernel(x, edge_index, batch, c1_init, c1_w, c1_root, c1_bias, c2_init, c2_w, c2_root, c2_bias, lin_w, lin_b):
    raise NotImplementedError("write your pallas kernel here")


# speedup vs baseline: 172.3719x; 172.3719x over previous
import functools

import jax
import jax.numpy as jnp
from jax import lax
from jax.experimental import pallas as pl
from jax.experimental.pallas import tpu as pltpu
from jax.experimental.pallas import tpu_sc as plsc

N = 50000
E = 800000
G = 128
CHUNK = 125
NCH = E // CHUNK
NB = 1000
NSTEPS = N // NB
BLKR = 1000
NBLK = N // BLKR
ZB0 = 40

f32 = jnp.float32
SDS = jax.ShapeDtypeStruct


def _mesh():
    return plsc.VectorSubcoreMesh(core_axis_name="c", subcore_axis_name="s")


_SC_PARAMS = pltpu.CompilerParams(use_tc_tiling_on_sc=False)


def _zero_blocks(sid, acc, zbuf):
    for i in range((NBLK + 15) // 16):
        bid = sid + 16 * i

        @pl.when(bid < NBLK)
        def _():
            base = pl.multiple_of(bid * BLKR, 8)

            def zc(j, carry):
                pltpu.sync_copy(zbuf, acc.at[pl.ds(base + j * ZB0, ZB0)])
                return carry

            lax.fori_loop(0, BLKR // ZB0, zc, 0)


def _writeback_blocks(sid, acc, out):
    for i in range((NBLK + 15) // 16):
        bid = sid + 16 * i

        @pl.when(bid < NBLK)
        def _():
            base = pl.multiple_of(bid * BLKR, 8)
            pltpu.sync_copy(acc.at[pl.ds(base, BLKR)],
                            out.at[pl.ds(base, BLKR)])


def _sc_degree(col2d):
    CPT = NCH // 32

    @functools.partial(
        pl.kernel,
        out_type=(SDS((N, 16), f32), SDS((N, 16), f32)),
        mesh=_mesh(),
        compiler_params=_SC_PARAMS,
        scratch_types=[
            pltpu.VMEM_SHARED((N, 16), f32),
            pltpu.VMEM((CPT, CHUNK), jnp.int32),
            pltpu.VMEM((CHUNK, 16), f32),
            pltpu.VMEM((ZB0, 16), f32),
        ],
    )
    def k(col_ref, out0, out1, acc, idxc, ones_v, zbuf):
        cid = lax.axis_index("c")
        sid = lax.axis_index("s")
        wid = cid * 16 + sid

        def fill(i, carry):
            ones_v[i, :] = jnp.ones((16,), f32)
            return carry

        lax.fori_loop(0, CHUNK, fill, 0)

        def zfill(i, carry):
            zbuf[i, :] = jnp.zeros((16,), f32)
            return carry

        lax.fori_loop(0, ZB0, zfill, 0)

        _zero_blocks(sid, acc, zbuf)
        plsc.subcore_barrier()

        pltpu.sync_copy(col_ref.at[pl.ds(wid * CPT, CPT)], idxc)

        def body(j, carry):
            pltpu.sync_copy(ones_v, acc.at[idxc.at[j]], add=True)
            return carry

        lax.fori_loop(0, CPT, body, 0)
        plsc.subcore_barrier()

        @pl.when(cid == 0)
        def _():
            _writeback_blocks(sid, acc, out0)

        @pl.when(cid == 1)
        def _():
            _writeback_blocks(sid, acc, out1)

    return k(col2d)


def _sc_prop2x32(x0, x1, row2d, col2d):
    CPT = NCH // 16
    GRP = 16
    NGRP = CPT // GRP

    @functools.partial(
        pl.kernel,
        out_type=(SDS((N, 32), f32), SDS((N, 32), f32)),
        mesh=_mesh(),
        compiler_params=_SC_PARAMS,
        scratch_types=[
            pltpu.VMEM_SHARED((N, 32), f32),
            pltpu.VMEM((GRP, CHUNK), jnp.int32),
            pltpu.VMEM((GRP, CHUNK), jnp.int32),
            pltpu.VMEM((CHUNK, 32), f32),
            pltpu.VMEM((ZB0, 32), f32),
        ],
    )
    def k(x0r, x1r, rowr, colr, out0, out1, acc, idxr, idxc, gbuf, zbuf):
        cid = lax.axis_index("c")
        sid = lax.axis_index("s")

        def zfill(i, carry):
            z = jnp.zeros((16,), f32)
            zbuf[i, pl.ds(0, 16)] = z
            zbuf[i, pl.ds(16, 16)] = z
            return carry

        lax.fori_loop(0, ZB0, zfill, 0)

        _zero_blocks(sid, acc, zbuf)
        plsc.subcore_barrier()

        def run(src):
            def grp(g, carry):
                base = pl.multiple_of(sid * CPT + g * GRP, 8)
                pltpu.sync_copy(rowr.at[pl.ds(base, GRP)], idxr)
                pltpu.sync_copy(colr.at[pl.ds(base, GRP)], idxc)

                def body(j, carry2):
                    pltpu.sync_copy(src.at[idxr.at[j]], gbuf)
                    pltpu.sync_copy(gbuf, acc.at[idxc.at[j]], add=True)
                    return carry2

                lax.fori_loop(0, GRP, body, 0)
                return carry

            lax.fori_loop(0, NGRP, grp, 0)

        @pl.when(cid == 0)
        def _():
            run(x0r)

        @pl.when(cid == 1)
        def _():
            run(x1r)

        plsc.subcore_barrier()

        @pl.when(cid == 0)
        def _():
            _writeback_blocks(sid, acc, out0)

        @pl.when(cid == 1)
        def _():
            _writeback_blocks(sid, acc, out1)

    return k(x0, x1, row2d, col2d)


def _sc_prop16(src, row2d, col2d):
    CPT = NCH // 32

    @functools.partial(
        pl.kernel,
        out_type=(SDS((N, 16), f32), SDS((N, 16), f32)),
        mesh=_mesh(),
        compiler_params=_SC_PARAMS,
        scratch_types=[
            pltpu.VMEM_SHARED((N, 16), f32),
            pltpu.VMEM((CPT, CHUNK), jnp.int32),
            pltpu.VMEM((CPT, CHUNK), jnp.int32),
            pltpu.VMEM((CHUNK, 16), f32),
            pltpu.VMEM((ZB0, 16), f32),
        ],
    )
    def k(srcr, rowr, colr, out0, out1, acc, idxr, idxc, gbuf, zbuf):
        cid = lax.axis_index("c")
        sid = lax.axis_index("s")
        wid = cid * 16 + sid

        def zfill(i, carry):
            zbuf[i, :] = jnp.zeros((16,), f32)
            return carry

        lax.fori_loop(0, ZB0, zfill, 0)

        _zero_blocks(sid, acc, zbuf)
        plsc.subcore_barrier()

        pltpu.sync_copy(rowr.at[pl.ds(wid * CPT, CPT)], idxr)
        pltpu.sync_copy(colr.at[pl.ds(wid * CPT, CPT)], idxc)

        def body(j, carry):
            pltpu.sync_copy(srcr.at[idxr.at[j]], gbuf)
            pltpu.sync_copy(gbuf, acc.at[idxc.at[j]], add=True)
            return carry

        lax.fori_loop(0, CPT, body, 0)
        plsc.subcore_barrier()

        @pl.when(cid == 0)
        def _():
            _writeback_blocks(sid, acc, out0)

        @pl.when(cid == 1)
        def _():
            _writeback_blocks(sid, acc, out1)

    return k(src, row2d, col2d)


_DOT = dict(preferred_element_type=f32, precision=lax.Precision.HIGHEST)


def _tc_a(x, d0, d1, i1s, r1s, b1v):
    def body(x_ref, d0_ref, d1_ref, w_ref, wr_ref, b_ref,
             dis_ref, xs0_ref, xs1_ref, rp_ref):
        deg = d0_ref[:, 0:1] + d1_ref[:, 0:1]
        dis = jnp.where(deg > 0.0, lax.rsqrt(jnp.maximum(deg, 1e-30)), 0.0)
        dis_ref[...] = dis
        xs = jnp.dot(x_ref[...], w_ref[...], **_DOT) * dis
        pad = jnp.zeros((NB, 8), f32)
        xs0_ref[...] = jnp.concatenate([xs[:, :24], pad], axis=1)
        xs1_ref[...] = jnp.concatenate([xs[:, 24:], pad], axis=1)
        rp_ref[...] = jnp.dot(x_ref[...], wr_ref[...], **_DOT) + b_ref[...]

    return pl.pallas_call(
        body,
        grid=(NSTEPS,),
        in_specs=[
            pl.BlockSpec((NB, 75), lambda i: (i, 0)),
            pl.BlockSpec((NB, 16), lambda i: (i, 0)),
            pl.BlockSpec((NB, 16), lambda i: (i, 0)),
            pl.BlockSpec((75, 48), lambda i: (0, 0)),
            pl.BlockSpec((75, 48), lambda i: (0, 0)),
            pl.BlockSpec((1, 48), lambda i: (0, 0)),
        ],
        out_specs=[
            pl.BlockSpec((NB, 1), lambda i: (i, 0)),
            pl.BlockSpec((NB, 32), lambda i: (i, 0)),
            pl.BlockSpec((NB, 32), lambda i: (i, 0)),
            pl.BlockSpec((NB, 48), lambda i: (i, 0)),
        ],
        out_shape=[SDS((N, 1), f32), SDS((N, 32), f32),
                   SDS((N, 32), f32), SDS((N, 48), f32)],
    )(x, d0, d1, i1s, r1s, b1v)


def _tc_b(p0, p1, rp, dis, w1bd):
    def body(p0_ref, p1_ref, rp_ref, dis_ref, w_ref, y0_ref, y1_ref):
        dis = dis_ref[...]
        p = jnp.concatenate([p0_ref[:, :24], p1_ref[:, :24]], axis=1)
        o1 = jnp.maximum(p * dis + rp_ref[...], 0.0)
        y = jnp.dot(o1, w_ref[...], **_DOT) * dis
        pad = jnp.zeros((NB, 8), f32)
        y0_ref[...] = jnp.concatenate([y[:, :24], pad], axis=1)
        y1_ref[...] = jnp.concatenate([y[:, 24:], pad], axis=1)

    return pl.pallas_call(
        body,
        grid=(NSTEPS,),
        in_specs=[
            pl.BlockSpec((NB, 32), lambda i: (i, 0)),
            pl.BlockSpec((NB, 32), lambda i: (i, 0)),
            pl.BlockSpec((NB, 48), lambda i: (i, 0)),
            pl.BlockSpec((NB, 1), lambda i: (i, 0)),
            pl.BlockSpec((48, 48), lambda i: (0, 0)),
        ],
        out_specs=[
            pl.BlockSpec((NB, 32), lambda i: (i, 0)),
            pl.BlockSpec((NB, 32), lambda i: (i, 0)),
        ],
        out_shape=[SDS((N, 32), f32), SDS((N, 32), f32)],
    )(p0, p1, rp, dis, w1bd)


def _tc_c(p0, p1, rp, dis, c2i, c2w0, c2r0, lint):
    def body(p0_ref, p1_ref, rp_ref, dis_ref, i2_ref, w2_ref, r2_ref,
             lin_ref, t_ref):
        dis = dis_ref[...]
        p = jnp.concatenate([p0_ref[:, :24], p1_ref[:, :24]], axis=1)
        o2 = jnp.maximum(p * dis + rp_ref[...], 0.0)
        h1 = (o2[:, :16] + o2[:, 16:32] + o2[:, 32:]) * (1.0 / 3.0)
        lint = lin_ref[...]
        ca, cb = [], []
        cdsum = jnp.zeros((16, 1), f32)
        for kk in range(3):
            wt = jnp.dot(w2_ref[kk], lint, **_DOT)
            ca.append(jnp.dot(i2_ref[kk], wt, **_DOT))
            cb.append(jnp.dot(r2_ref[kk], wt, **_DOT))
            cdsum = cdsum + jnp.dot(r2_ref[kk], lint, **_DOT)
        a = jnp.dot(h1, jnp.concatenate(ca, axis=1), **_DOT)
        b = jnp.dot(h1, jnp.concatenate(cb, axis=1), **_DOT)
        dsum = jnp.dot(h1, cdsum, **_DOT)
        pad = jnp.zeros((NB, 8), f32)
        t_ref[...] = jnp.concatenate(
            [a * dis, b * dis, dis, dsum, pad], axis=1)

    return pl.pallas_call(
        body,
        grid=(NSTEPS,),
        in_specs=[
            pl.BlockSpec((NB, 32), lambda i: (i, 0)),
            pl.BlockSpec((NB, 32), lambda i: (i, 0)),
            pl.BlockSpec((NB, 48), lambda i: (i, 0)),
            pl.BlockSpec((NB, 1), lambda i: (i, 0)),
            pl.BlockSpec((3, 16, 64), lambda i: (0, 0, 0)),
            pl.BlockSpec((3, 64, 64), lambda i: (0, 0, 0)),
            pl.BlockSpec((3, 16, 64), lambda i: (0, 0, 0)),
            pl.BlockSpec((64, 1), lambda i: (0, 0)),
        ],
        out_specs=pl.BlockSpec((NB, 16), lambda i: (i, 0)),
        out_shape=SDS((N, 16), f32),
    )(p0, p1, rp, dis, c2i, c2w0, c2r0, lint)


def _tc_d(q0, q1, t, dis, c2w0, c2b, lint):
    def body(q0_ref, q1_ref, t_ref, dis_ref, w2_ref, b2_ref, lin_ref, u_ref):
        dis = dis_ref[...]
        acc3 = q0_ref[...] + q1_ref[...]
        lint = lin_ref[...]
        b1s = jnp.zeros((1, 1), f32)
        for kk in range(3):
            wt = jnp.dot(w2_ref[kk], lint, **_DOT)
            b1s = b1s + jnp.dot(b2_ref[kk:kk + 1, :], wt, **_DOT)
        src2 = acc3[:, 0:3] * dis * dis
        spart = (acc3[:, 3:4] + acc3[:, 4:5] + acc3[:, 5:6]
                 + b1s * acc3[:, 6:7]) * dis + t_ref[:, 7:8]
        pad = jnp.zeros((NB, 12), f32)
        u_ref[...] = jnp.concatenate([src2, spart, pad], axis=1)

    return pl.pallas_call(
        body,
        grid=(NSTEPS,),
        in_specs=[
            pl.BlockSpec((NB, 16), lambda i: (i, 0)),
            pl.BlockSpec((NB, 16), lambda i: (i, 0)),
            pl.BlockSpec((NB, 16), lambda i: (i, 0)),
            pl.BlockSpec((NB, 1), lambda i: (i, 0)),
            pl.BlockSpec((3, 64, 64), lambda i: (0, 0, 0)),
            pl.BlockSpec((3, 64), lambda i: (0, 0)),
            pl.BlockSpec((64, 1), lambda i: (0, 0)),
        ],
        out_specs=pl.BlockSpec((NB, 16), lambda i: (i, 0)),
        out_shape=SDS((N, 16), f32),
    )(q0, q1, t, dis, c2w0, c2b, lint)


def _tc_e(r0, r1, u, dis, batch2d, c2b, lint, linb):
    def body(r0_ref, r1_ref, u_ref, dis_ref, bt_ref, b2_ref, lin_ref,
             lb_ref, z_ref, zacc, nacc):
        i = pl.program_id(0)
        acc4 = r0_ref[...] + r1_ref[...]
        s = (acc4[:, 0:1] + acc4[:, 1:2] + acc4[:, 2:3]) * dis_ref[...] \
            + u_ref[:, 3:4]
        iota = lax.broadcasted_iota(jnp.int32, (1, G), 1)
        m = (bt_ref[...] == iota).astype(f32)
        zp = jnp.sum(m * s, axis=0, keepdims=True)
        npp = jnp.sum(m, axis=0, keepdims=True)

        @pl.when(i == 0)
        def _():
            zacc[...] = jnp.zeros((1, G), f32)
            nacc[...] = jnp.zeros((1, G), f32)

        zacc[...] += zp
        nacc[...] += npp

        @pl.when(i == NSTEPS - 1)
        def _():
            lint = lin_ref[...]
            b2s = jnp.zeros((1, 1), f32)
            for kk in range(3):
                b2s = b2s + jnp.dot(b2_ref[kk:kk + 1, :], lint, **_DOT)
            z_ref[...] = (zacc[...] + b2s * nacc[...]) * (1.0 / 3.0) \
                + lb_ref[...]

    return pl.pallas_call(
        body,
        grid=(NSTEPS,),
        in_specs=[
            pl.BlockSpec((NB, 16), lambda i: (i, 0)),
            pl.BlockSpec((NB, 16), lambda i: (i, 0)),
            pl.BlockSpec((NB, 16), lambda i: (i, 0)),
            pl.BlockSpec((NB, 1), lambda i: (i, 0)),
            pl.BlockSpec((NB, 1), lambda i: (i, 0)),
            pl.BlockSpec((3, 64), lambda i: (0, 0)),
            pl.BlockSpec((64, 1), lambda i: (0, 0)),
            pl.BlockSpec((1, 1), lambda i: (0, 0)),
        ],
        out_specs=pl.BlockSpec((1, G), lambda i: (0, 0)),
        out_shape=SDS((1, G), f32),
        scratch_shapes=[pltpu.VMEM((1, G), f32), pltpu.VMEM((1, G), f32)],
    )(r0, r1, u, dis, batch2d, c2b, lint, linb)


def kernel(x, edge_index, batch, c1_init, c1_w, c1_root, c1_bias,
           c2_init, c2_w, c2_root, c2_bias, lin_w, lin_b):
    row2d = edge_index[0].reshape(NCH, CHUNK)
    col2d = edge_index[1].reshape(NCH, CHUNK)
    batch2d = batch.reshape(N, 1)

    i1s = c1_init.transpose(1, 0, 2).reshape(75, 48)
    r1s = c1_root[0].transpose(1, 0, 2).reshape(75, 48)
    b1v = c1_bias[0].reshape(1, 48)
    w1bd = jnp.zeros((48, 48), f32)
    for kk in range(3):
        w1bd = w1bd.at[kk * 16:(kk + 1) * 16, kk * 16:(kk + 1) * 16].set(
            c1_w[0, kk])
    c2w0 = c2_w[0]
    c2r0 = c2_root[0]
    c2b = c2_bias[0].reshape(3, 64)
    lint = lin_w.reshape(64, 1)
    linb = lin_b.reshape(1, 1)

    d0, d1 = _sc_degree(col2d)
    dis, xs0, xs1, rp = _tc_a(x, d0, d1, i1s, r1s, b1v)
    p10, p11 = _sc_prop2x32(xs0, xs1, row2d, col2d)
    ys0, ys1 = _tc_b(p10, p11, rp, dis, w1bd)
    p20, p21 = _sc_prop2x32(ys0, ys1, row2d, col2d)
    t = _tc_c(p20, p21, rp, dis, c2_init, c2w0, c2r0, lint)
    q0, q1 = _sc_prop16(t, row2d, col2d)
    u = _tc_d(q0, q1, t, dis, c2w0, c2b, lint)
    r0, r1 = _sc_prop16(u, row2d, col2d)
    zrow = _tc_e(r0, r1, u, dis, batch2d, c2b, lint, linb)
    return zrow.reshape(G, 1)


# speedup vs baseline: 279.4035x; 1.6209x over previous
import functools

import jax
import jax.numpy as jnp
from jax import lax
from jax.experimental import pallas as pl
from jax.experimental.pallas import tpu as pltpu
from jax.experimental.pallas import tpu_sc as plsc

N = 50000
E = 800000
G = 128
CHUNK = 125
NCH = E // CHUNK
NB = 1000
NSTEPS = N // NB
BLKR = 1000
NBLK = N // BLKR
ZB0 = 40
NBUF = 4
GRPC = 40

f32 = jnp.float32
SDS = jax.ShapeDtypeStruct


def _mesh():
    return plsc.VectorSubcoreMesh(core_axis_name="c", subcore_axis_name="s")


_SC_PARAMS = pltpu.CompilerParams(use_tc_tiling_on_sc=False)


def _zero_blocks(sid, acc, zbuf, sem):
    for i in range((NBLK + 15) // 16):
        bid = sid + 16 * i

        @pl.when(bid < NBLK)
        def _():
            base = pl.multiple_of(bid * BLKR, 8)
            ds = [pltpu.async_copy(zbuf, acc.at[pl.ds(base + j * ZB0, ZB0)],
                                   sem)
                  for j in range(BLKR // ZB0)]
            for d in ds:
                d.wait()


def _writeback_blocks(sid, acc, out, sem):
    ds = []
    for i in range((NBLK + 15) // 16):
        bid = sid + 16 * i

        @pl.when(bid < NBLK)
        def _():
            base = pl.multiple_of(bid * BLKR, 8)
            pltpu.async_copy(acc.at[pl.ds(base, BLKR)],
                             out.at[pl.ds(base, BLKR)], sem).wait()


def _sc_degree(col2d):
    CPT = NCH // 32
    NGRP = CPT // GRPC

    @functools.partial(
        pl.kernel,
        out_type=(SDS((N, 16), f32), SDS((N, 16), f32)),
        mesh=_mesh(),
        compiler_params=_SC_PARAMS,
        scratch_types=[
            pltpu.VMEM_SHARED((N, 16), f32),
            pltpu.VMEM((GRPC, CHUNK), jnp.int32),
            pltpu.VMEM((CHUNK, 16), f32),
            pltpu.VMEM((ZB0, 16), f32),
            pltpu.SemaphoreType.DMA,
            pltpu.SemaphoreType.DMA,
        ],
    )
    def k(col_ref, out0, out1, acc, idxc, ones_v, zbuf, ssem, isem):
        cid = lax.axis_index("c")
        sid = lax.axis_index("s")
        wid = cid * 16 + sid

        def fill(i, carry):
            ones_v[i, :] = jnp.ones((16,), f32)
            return carry

        lax.fori_loop(0, CHUNK, fill, 0)

        def zfill(i, carry):
            zbuf[i, :] = jnp.zeros((16,), f32)
            return carry

        lax.fori_loop(0, ZB0, zfill, 0)

        _zero_blocks(sid, acc, zbuf, ssem)
        plsc.subcore_barrier()

        def group(g, carry):
            ibase = pl.multiple_of(wid * CPT + g * GRPC, 8)
            pltpu.async_copy(col_ref.at[pl.ds(ibase, GRPC)], idxc, isem).wait()
            scs = []
            for c in range(GRPC):
                scs.append(pltpu.async_copy(
                    ones_v, acc.at[idxc.at[c]], ssem, add=True))
                if c >= 3:
                    scs[c - 3].wait()
            for c in range(GRPC - 3, GRPC):
                scs[c].wait()
            return carry

        lax.fori_loop(0, NGRP, group, 0)
        plsc.subcore_barrier()

        @pl.when(cid == 0)
        def _():
            _writeback_blocks(sid, acc, out0, ssem)

        @pl.when(cid == 1)
        def _():
            _writeback_blocks(sid, acc, out1, ssem)

    return k(col2d)


def _sc_prop2x32(x0, x1, row2d, col2d):
    CPT = NCH // 16
    NGRP = CPT // GRPC

    @functools.partial(
        pl.kernel,
        out_type=(SDS((N, 32), f32), SDS((N, 32), f32)),
        mesh=_mesh(),
        compiler_params=_SC_PARAMS,
        scratch_types=[
            pltpu.VMEM_SHARED((N, 32), f32),
            pltpu.VMEM((GRPC, CHUNK), jnp.int32),
            pltpu.VMEM((GRPC, CHUNK), jnp.int32),
            pltpu.VMEM((NBUF, CHUNK, 32), f32),
            pltpu.VMEM((ZB0, 32), f32),
            pltpu.SemaphoreType.DMA,
            pltpu.SemaphoreType.DMA,
            pltpu.SemaphoreType.DMA,
        ],
    )
    def k(x0r, x1r, rowr, colr, out0, out1, acc, idxr, idxc, gbuf, zbuf,
          gsem, ssem, isem):
        cid = lax.axis_index("c")
        sid = lax.axis_index("s")

        def zfill(i, carry):
            z = jnp.zeros((16,), f32)
            zbuf[i, pl.ds(0, 16)] = z
            zbuf[i, pl.ds(16, 16)] = z
            return carry

        lax.fori_loop(0, ZB0, zfill, 0)

        _zero_blocks(sid, acc, zbuf, ssem)
        plsc.subcore_barrier()

        def run(src):
            def group(g, carry):
                ibase = pl.multiple_of(sid * CPT + g * GRPC, 8)
                ir = pltpu.async_copy(rowr.at[pl.ds(ibase, GRPC)], idxr, isem)
                ic = pltpu.async_copy(colr.at[pl.ds(ibase, GRPC)], idxc, isem)
                ir.wait()
                ic.wait()
                gds = [pltpu.async_copy(src.at[idxr.at[c]],
                                        gbuf.at[c % NBUF], gsem)
                       for c in range(NBUF - 1)]
                scs = []
                for c in range(GRPC):
                    if c >= 1:
                        scs[c - 1].wait()
                    if c + NBUF - 1 < GRPC:
                        gds.append(pltpu.async_copy(
                            src.at[idxr.at[c + NBUF - 1]],
                            gbuf.at[(c + NBUF - 1) % NBUF], gsem))
                    gds[c].wait()
                    scs.append(pltpu.async_copy(
                        gbuf.at[c % NBUF], acc.at[idxc.at[c]], ssem,
                        add=True))
                scs[GRPC - 1].wait()
                return carry

            lax.fori_loop(0, NGRP, group, 0)

        @pl.when(cid == 0)
        def _():
            run(x0r)

        @pl.when(cid == 1)
        def _():
            run(x1r)

        plsc.subcore_barrier()

        @pl.when(cid == 0)
        def _():
            _writeback_blocks(sid, acc, out0, ssem)

        @pl.when(cid == 1)
        def _():
            _writeback_blocks(sid, acc, out1, ssem)

    return k(x0, x1, row2d, col2d)


def _sc_prop16(src, row2d, col2d):
    CPT = NCH // 32
    NGRP = CPT // GRPC

    @functools.partial(
        pl.kernel,
        out_type=(SDS((N, 16), f32), SDS((N, 16), f32)),
        mesh=_mesh(),
        compiler_params=_SC_PARAMS,
        scratch_types=[
            pltpu.VMEM_SHARED((N, 16), f32),
            pltpu.VMEM((GRPC, CHUNK), jnp.int32),
            pltpu.VMEM((GRPC, CHUNK), jnp.int32),
            pltpu.VMEM((NBUF, CHUNK, 16), f32),
            pltpu.VMEM((ZB0, 16), f32),
            pltpu.SemaphoreType.DMA,
            pltpu.SemaphoreType.DMA,
            pltpu.SemaphoreType.DMA,
        ],
    )
    def k(srcr, rowr, colr, out0, out1, acc, idxr, idxc, gbuf, zbuf,
          gsem, ssem, isem):
        cid = lax.axis_index("c")
        sid = lax.axis_index("s")
        wid = cid * 16 + sid

        def zfill(i, carry):
            zbuf[i, :] = jnp.zeros((16,), f32)
            return carry

        lax.fori_loop(0, ZB0, zfill, 0)

        _zero_blocks(sid, acc, zbuf, ssem)
        plsc.subcore_barrier()

        def group(g, carry):
            ibase = pl.multiple_of(wid * CPT + g * GRPC, 8)
            ir = pltpu.async_copy(rowr.at[pl.ds(ibase, GRPC)], idxr, isem)
            ic = pltpu.async_copy(colr.at[pl.ds(ibase, GRPC)], idxc, isem)
            ir.wait()
            ic.wait()
            gds = [pltpu.async_copy(srcr.at[idxr.at[c]],
                                    gbuf.at[c % NBUF], gsem)
                   for c in range(NBUF - 1)]
            scs = []
            for c in range(GRPC):
                if c >= 1:
                    scs[c - 1].wait()
                if c + NBUF - 1 < GRPC:
                    gds.append(pltpu.async_copy(
                        srcr.at[idxr.at[c + NBUF - 1]],
                        gbuf.at[(c + NBUF - 1) % NBUF], gsem))
                gds[c].wait()
                scs.append(pltpu.async_copy(
                    gbuf.at[c % NBUF], acc.at[idxc.at[c]], ssem, add=True))
            scs[GRPC - 1].wait()
            return carry

        lax.fori_loop(0, NGRP, group, 0)
        plsc.subcore_barrier()

        @pl.when(cid == 0)
        def _():
            _writeback_blocks(sid, acc, out0, ssem)

        @pl.when(cid == 1)
        def _():
            _writeback_blocks(sid, acc, out1, ssem)

    return k(src, row2d, col2d)


_DOT = dict(preferred_element_type=f32, precision=lax.Precision.HIGHEST)


def _tc_a(x, d0, d1, i1s, r1s, b1v):
    def body(x_ref, d0_ref, d1_ref, w_ref, wr_ref, b_ref,
             dis_ref, xs0_ref, xs1_ref, rp_ref):
        deg = d0_ref[:, 0:1] + d1_ref[:, 0:1]
        dis = jnp.where(deg > 0.0, lax.rsqrt(jnp.maximum(deg, 1e-30)), 0.0)
        dis_ref[...] = dis
        xs = jnp.dot(x_ref[...], w_ref[...], **_DOT) * dis
        pad = jnp.zeros((NB, 8), f32)
        xs0_ref[...] = jnp.concatenate([xs[:, :24], pad], axis=1)
        xs1_ref[...] = jnp.concatenate([xs[:, 24:], pad], axis=1)
        rp_ref[...] = jnp.dot(x_ref[...], wr_ref[...], **_DOT) + b_ref[...]

    return pl.pallas_call(
        body,
        grid=(NSTEPS,),
        in_specs=[
            pl.BlockSpec((NB, 75), lambda i: (i, 0)),
            pl.BlockSpec((NB, 16), lambda i: (i, 0)),
            pl.BlockSpec((NB, 16), lambda i: (i, 0)),
            pl.BlockSpec((75, 48), lambda i: (0, 0)),
            pl.BlockSpec((75, 48), lambda i: (0, 0)),
            pl.BlockSpec((1, 48), lambda i: (0, 0)),
        ],
        out_specs=[
            pl.BlockSpec((NB, 1), lambda i: (i, 0)),
            pl.BlockSpec((NB, 32), lambda i: (i, 0)),
            pl.BlockSpec((NB, 32), lambda i: (i, 0)),
            pl.BlockSpec((NB, 48), lambda i: (i, 0)),
        ],
        out_shape=[SDS((N, 1), f32), SDS((N, 32), f32),
                   SDS((N, 32), f32), SDS((N, 48), f32)],
    )(x, d0, d1, i1s, r1s, b1v)


def _tc_b(p0, p1, rp, dis, w1bd):
    def body(p0_ref, p1_ref, rp_ref, dis_ref, w_ref, y0_ref, y1_ref):
        dis = dis_ref[...]
        p = jnp.concatenate([p0_ref[:, :24], p1_ref[:, :24]], axis=1)
        o1 = jnp.maximum(p * dis + rp_ref[...], 0.0)
        y = jnp.dot(o1, w_ref[...], **_DOT) * dis
        pad = jnp.zeros((NB, 8), f32)
        y0_ref[...] = jnp.concatenate([y[:, :24], pad], axis=1)
        y1_ref[...] = jnp.concatenate([y[:, 24:], pad], axis=1)

    return pl.pallas_call(
        body,
        grid=(NSTEPS,),
        in_specs=[
            pl.BlockSpec((NB, 32), lambda i: (i, 0)),
            pl.BlockSpec((NB, 32), lambda i: (i, 0)),
            pl.BlockSpec((NB, 48), lambda i: (i, 0)),
            pl.BlockSpec((NB, 1), lambda i: (i, 0)),
            pl.BlockSpec((48, 48), lambda i: (0, 0)),
        ],
        out_specs=[
            pl.BlockSpec((NB, 32), lambda i: (i, 0)),
            pl.BlockSpec((NB, 32), lambda i: (i, 0)),
        ],
        out_shape=[SDS((N, 32), f32), SDS((N, 32), f32)],
    )(p0, p1, rp, dis, w1bd)


def _tc_c(p0, p1, rp, dis, c2i, c2w0, c2r0, lint):
    def body(p0_ref, p1_ref, rp_ref, dis_ref, i2_ref, w2_ref, r2_ref,
             lin_ref, t_ref):
        dis = dis_ref[...]
        p = jnp.concatenate([p0_ref[:, :24], p1_ref[:, :24]], axis=1)
        o2 = jnp.maximum(p * dis + rp_ref[...], 0.0)
        h1 = (o2[:, :16] + o2[:, 16:32] + o2[:, 32:]) * (1.0 / 3.0)
        lint = lin_ref[...]
        ca, cb = [], []
        cdsum = jnp.zeros((16, 1), f32)
        for kk in range(3):
            wt = jnp.dot(w2_ref[kk], lint, **_DOT)
            ca.append(jnp.dot(i2_ref[kk], wt, **_DOT))
            cb.append(jnp.dot(r2_ref[kk], wt, **_DOT))
            cdsum = cdsum + jnp.dot(r2_ref[kk], lint, **_DOT)
        a = jnp.dot(h1, jnp.concatenate(ca, axis=1), **_DOT)
        b = jnp.dot(h1, jnp.concatenate(cb, axis=1), **_DOT)
        dsum = jnp.dot(h1, cdsum, **_DOT)
        pad = jnp.zeros((NB, 8), f32)
        t_ref[...] = jnp.concatenate(
            [a * dis, b * dis, dis, dsum, pad], axis=1)

    return pl.pallas_call(
        body,
        grid=(NSTEPS,),
        in_specs=[
            pl.BlockSpec((NB, 32), lambda i: (i, 0)),
            pl.BlockSpec((NB, 32), lambda i: (i, 0)),
            pl.BlockSpec((NB, 48), lambda i: (i, 0)),
            pl.BlockSpec((NB, 1), lambda i: (i, 0)),
            pl.BlockSpec((3, 16, 64), lambda i: (0, 0, 0)),
            pl.BlockSpec((3, 64, 64), lambda i: (0, 0, 0)),
            pl.BlockSpec((3, 16, 64), lambda i: (0, 0, 0)),
            pl.BlockSpec((64, 1), lambda i: (0, 0)),
        ],
        out_specs=pl.BlockSpec((NB, 16), lambda i: (i, 0)),
        out_shape=SDS((N, 16), f32),
    )(p0, p1, rp, dis, c2i, c2w0, c2r0, lint)


def _tc_d(q0, q1, t, dis, c2w0, c2b, lint):
    def body(q0_ref, q1_ref, t_ref, dis_ref, w2_ref, b2_ref, lin_ref, u_ref):
        dis = dis_ref[...]
        acc3 = q0_ref[...] + q1_ref[...]
        lint = lin_ref[...]
        b1s = jnp.zeros((1, 1), f32)
        for kk in range(3):
            wt = jnp.dot(w2_ref[kk], lint, **_DOT)
            b1s = b1s + jnp.dot(b2_ref[kk:kk + 1, :], wt, **_DOT)
        src2 = acc3[:, 0:3] * dis * dis
        spart = (acc3[:, 3:4] + acc3[:, 4:5] + acc3[:, 5:6]
                 + b1s * acc3[:, 6:7]) * dis + t_ref[:, 7:8]
        pad = jnp.zeros((NB, 12), f32)
        u_ref[...] = jnp.concatenate([src2, spart, pad], axis=1)

    return pl.pallas_call(
        body,
        grid=(NSTEPS,),
        in_specs=[
            pl.BlockSpec((NB, 16), lambda i: (i, 0)),
            pl.BlockSpec((NB, 16), lambda i: (i, 0)),
            pl.BlockSpec((NB, 16), lambda i: (i, 0)),
            pl.BlockSpec((NB, 1), lambda i: (i, 0)),
            pl.BlockSpec((3, 64, 64), lambda i: (0, 0, 0)),
            pl.BlockSpec((3, 64), lambda i: (0, 0)),
            pl.BlockSpec((64, 1), lambda i: (0, 0)),
        ],
        out_specs=pl.BlockSpec((NB, 16), lambda i: (i, 0)),
        out_shape=SDS((N, 16), f32),
    )(q0, q1, t, dis, c2w0, c2b, lint)


def _tc_e(r0, r1, u, dis, batch2d, c2b, lint, linb):
    def body(r0_ref, r1_ref, u_ref, dis_ref, bt_ref, b2_ref, lin_ref,
             lb_ref, z_ref, zacc, nacc):
        i = pl.program_id(0)
        acc4 = r0_ref[...] + r1_ref[...]
        s = (acc4[:, 0:1] + acc4[:, 1:2] + acc4[:, 2:3]) * dis_ref[...] \
            + u_ref[:, 3:4]
        iota = lax.broadcasted_iota(jnp.int32, (1, G), 1)
        m = (bt_ref[...] == iota).astype(f32)
        zp = jnp.sum(m * s, axis=0, keepdims=True)
        npp = jnp.sum(m, axis=0, keepdims=True)

        @pl.when(i == 0)
        def _():
            zacc[...] = jnp.zeros((1, G), f32)
            nacc[...] = jnp.zeros((1, G), f32)

        zacc[...] += zp
        nacc[...] += npp

        @pl.when(i == NSTEPS - 1)
        def _():
            lint = lin_ref[...]
            b2s = jnp.zeros((1, 1), f32)
            for kk in range(3):
                b2s = b2s + jnp.dot(b2_ref[kk:kk + 1, :], lint, **_DOT)
            z_ref[...] = (zacc[...] + b2s * nacc[...]) * (1.0 / 3.0) \
                + lb_ref[...]

    return pl.pallas_call(
        body,
        grid=(NSTEPS,),
        in_specs=[
            pl.BlockSpec((NB, 16), lambda i: (i, 0)),
            pl.BlockSpec((NB, 16), lambda i: (i, 0)),
            pl.BlockSpec((NB, 16), lambda i: (i, 0)),
            pl.BlockSpec((NB, 1), lambda i: (i, 0)),
            pl.BlockSpec((NB, 1), lambda i: (i, 0)),
            pl.BlockSpec((3, 64), lambda i: (0, 0)),
            pl.BlockSpec((64, 1), lambda i: (0, 0)),
            pl.BlockSpec((1, 1), lambda i: (0, 0)),
        ],
        out_specs=pl.BlockSpec((1, G), lambda i: (0, 0)),
        out_shape=SDS((1, G), f32),
        scratch_shapes=[pltpu.VMEM((1, G), f32), pltpu.VMEM((1, G), f32)],
    )(r0, r1, u, dis, batch2d, c2b, lint, linb)


def kernel(x, edge_index, batch, c1_init, c1_w, c1_root, c1_bias,
           c2_init, c2_w, c2_root, c2_bias, lin_w, lin_b):
    row2d = edge_index[0].reshape(NCH, CHUNK)
    col2d = edge_index[1].reshape(NCH, CHUNK)
    batch2d = batch.reshape(N, 1)

    i1s = c1_init.transpose(1, 0, 2).reshape(75, 48)
    r1s = c1_root[0].transpose(1, 0, 2).reshape(75, 48)
    b1v = c1_bias[0].reshape(1, 48)
    w1bd = jnp.zeros((48, 48), f32)
    for kk in range(3):
        w1bd = w1bd.at[kk * 16:(kk + 1) * 16, kk * 16:(kk + 1) * 16].set(
            c1_w[0, kk])
    c2w0 = c2_w[0]
    c2r0 = c2_root[0]
    c2b = c2_bias[0].reshape(3, 64)
    lint = lin_w.reshape(64, 1)
    linb = lin_b.reshape(1, 1)

    d0, d1 = _sc_degree(col2d)
    dis, xs0, xs1, rp = _tc_a(x, d0, d1, i1s, r1s, b1v)
    p10, p11 = _sc_prop2x32(xs0, xs1, row2d, col2d)
    ys0, ys1 = _tc_b(p10, p11, rp, dis, w1bd)
    p20, p21 = _sc_prop2x32(ys0, ys1, row2d, col2d)
    t = _tc_c(p20, p21, rp, dis, c2_init, c2w0, c2r0, lint)
    q0, q1 = _sc_prop16(t, row2d, col2d)
    u = _tc_d(q0, q1, t, dis, c2w0, c2b, lint)
    r0, r1 = _sc_prop16(u, row2d, col2d)
    zrow = _tc_e(r0, r1, u, dis, batch2d, c2b, lint, linb)
    return zrow.reshape(G, 1)


# speedup vs baseline: 344.2871x; 1.2322x over previous
import functools

import jax
import jax.numpy as jnp
from jax import lax
from jax.experimental import pallas as pl
from jax.experimental.pallas import tpu as pltpu
from jax.experimental.pallas import tpu_sc as plsc

N = 50000
E = 800000
G = 128
CHUNK = 125
NCH = E // CHUNK
NB = 2000
NSTEPS = N // NB
BLKR = 1000
NBLK = N // BLKR
ZB0 = 40
NBUF = 4
NBUF16 = 6
GRPC = 40

f32 = jnp.float32
SDS = jax.ShapeDtypeStruct


def _mesh():
    return plsc.VectorSubcoreMesh(core_axis_name="c", subcore_axis_name="s")


_SC_PARAMS = pltpu.CompilerParams(use_tc_tiling_on_sc=False)


def _zero_blocks(sid, acc, zbuf, sem):
    for i in range((NBLK + 15) // 16):
        bid = sid + 16 * i

        @pl.when(bid < NBLK)
        def _():
            base = pl.multiple_of(bid * BLKR, 8)
            ds = [pltpu.async_copy(zbuf, acc.at[pl.ds(base + j * ZB0, ZB0)],
                                   sem)
                  for j in range(BLKR // ZB0)]
            for d in ds:
                d.wait()


def _writeback_blocks(sid, acc, out, sem):
    for i in range((NBLK + 15) // 16):
        bid = sid + 16 * i

        @pl.when(bid < NBLK)
        def _():
            base = pl.multiple_of(bid * BLKR, 8)
            pltpu.async_copy(acc.at[pl.ds(base, BLKR)],
                             out.at[pl.ds(base, BLKR)], sem).wait()


def _sc_degree(ei3):
    CPT = NCH // 32
    NGRP = CPT // GRPC

    @functools.partial(
        pl.kernel,
        out_type=(SDS((N, 16), f32), SDS((N, 16), f32)),
        mesh=_mesh(),
        compiler_params=_SC_PARAMS,
        scratch_types=[
            pltpu.VMEM_SHARED((N, 16), f32),
            pltpu.VMEM((GRPC, CHUNK), jnp.int32),
            pltpu.VMEM((CHUNK, 16), f32),
            pltpu.VMEM((ZB0, 16), f32),
            pltpu.SemaphoreType.DMA,
            pltpu.SemaphoreType.DMA,
        ],
    )
    def k(ei_ref, out0, out1, acc, idxc, ones_v, zbuf, ssem, isem):
        cid = lax.axis_index("c")
        sid = lax.axis_index("s")
        wid = cid * 16 + sid

        def fill(i, carry):
            ones_v[i, :] = jnp.ones((16,), f32)
            return carry

        lax.fori_loop(0, CHUNK, fill, 0)

        def zfill(i, carry):
            zbuf[i, :] = jnp.zeros((16,), f32)
            return carry

        lax.fori_loop(0, ZB0, zfill, 0)

        _zero_blocks(sid, acc, zbuf, ssem)
        plsc.subcore_barrier()

        def group(g, carry):
            ibase = pl.multiple_of(wid * CPT + g * GRPC, 8)
            pltpu.async_copy(ei_ref.at[1, pl.ds(ibase, GRPC)], idxc,
                             isem).wait()
            scs = []
            for c in range(GRPC):
                scs.append(pltpu.async_copy(
                    ones_v, acc.at[idxc.at[c]], ssem, add=True))
                if c >= 3:
                    scs[c - 3].wait()
            for c in range(GRPC - 3, GRPC):
                scs[c].wait()
            return carry

        lax.fori_loop(0, NGRP, group, 0)
        plsc.subcore_barrier()

        @pl.when(cid == 0)
        def _():
            _writeback_blocks(sid, acc, out0, ssem)

        @pl.when(cid == 1)
        def _():
            _writeback_blocks(sid, acc, out1, ssem)

    return k(ei3)


def _sc_prop2x32(x0, x1, ei3):
    CPT = NCH // 16
    NGRP = CPT // GRPC

    @functools.partial(
        pl.kernel,
        out_type=(SDS((N, 32), f32), SDS((N, 32), f32)),
        mesh=_mesh(),
        compiler_params=_SC_PARAMS,
        scratch_types=[
            pltpu.VMEM_SHARED((N, 32), f32),
            pltpu.VMEM((GRPC, CHUNK), jnp.int32),
            pltpu.VMEM((GRPC, CHUNK), jnp.int32),
            pltpu.VMEM((NBUF, CHUNK, 32), f32),
            pltpu.VMEM((ZB0, 32), f32),
            pltpu.SemaphoreType.DMA,
            pltpu.SemaphoreType.DMA,
            pltpu.SemaphoreType.DMA,
        ],
    )
    def k(x0r, x1r, ei_ref, out0, out1, acc, idxr, idxc, gbuf, zbuf,
          gsem, ssem, isem):
        cid = lax.axis_index("c")
        sid = lax.axis_index("s")

        def zfill(i, carry):
            z = jnp.zeros((16,), f32)
            zbuf[i, pl.ds(0, 16)] = z
            zbuf[i, pl.ds(16, 16)] = z
            return carry

        lax.fori_loop(0, ZB0, zfill, 0)

        _zero_blocks(sid, acc, zbuf, ssem)
        plsc.subcore_barrier()

        def run(src):
            def group(g, carry):
                ibase = pl.multiple_of(sid * CPT + g * GRPC, 8)
                ir = pltpu.async_copy(ei_ref.at[0, pl.ds(ibase, GRPC)],
                                      idxr, isem)
                ic = pltpu.async_copy(ei_ref.at[1, pl.ds(ibase, GRPC)],
                                      idxc, isem)
                ir.wait()
                ic.wait()
                gds = [pltpu.async_copy(src.at[idxr.at[c]],
                                        gbuf.at[c % NBUF], gsem)
                       for c in range(NBUF - 1)]
                scs = []
                for c in range(GRPC):
                    if c >= 1:
                        scs[c - 1].wait()
                    if c + NBUF - 1 < GRPC:
                        gds.append(pltpu.async_copy(
                            src.at[idxr.at[c + NBUF - 1]],
                            gbuf.at[(c + NBUF - 1) % NBUF], gsem))
                    gds[c].wait()
                    scs.append(pltpu.async_copy(
                        gbuf.at[c % NBUF], acc.at[idxc.at[c]], ssem,
                        add=True))
                scs[GRPC - 1].wait()
                return carry

            lax.fori_loop(0, NGRP, group, 0)

        @pl.when(cid == 0)
        def _():
            run(x0r)

        @pl.when(cid == 1)
        def _():
            run(x1r)

        plsc.subcore_barrier()

        @pl.when(cid == 0)
        def _():
            _writeback_blocks(sid, acc, out0, ssem)

        @pl.when(cid == 1)
        def _():
            _writeback_blocks(sid, acc, out1, ssem)

    return k(x0, x1, ei3)


def _sc_prop16(src, ei3):
    CPT = NCH // 32
    NGRP = CPT // GRPC

    @functools.partial(
        pl.kernel,
        out_type=(SDS((N, 16), f32), SDS((N, 16), f32)),
        mesh=_mesh(),
        compiler_params=_SC_PARAMS,
        scratch_types=[
            pltpu.VMEM_SHARED((N, 16), f32),
            pltpu.VMEM((GRPC, CHUNK), jnp.int32),
            pltpu.VMEM((GRPC, CHUNK), jnp.int32),
            pltpu.VMEM((NBUF16, CHUNK, 16), f32),
            pltpu.VMEM((ZB0, 16), f32),
            pltpu.SemaphoreType.DMA,
            pltpu.SemaphoreType.DMA,
            pltpu.SemaphoreType.DMA,
        ],
    )
    def k(srcr, ei_ref, out0, out1, acc, idxr, idxc, gbuf, zbuf,
          gsem, ssem, isem):
        cid = lax.axis_index("c")
        sid = lax.axis_index("s")
        wid = cid * 16 + sid

        def zfill(i, carry):
            zbuf[i, :] = jnp.zeros((16,), f32)
            return carry

        lax.fori_loop(0, ZB0, zfill, 0)

        _zero_blocks(sid, acc, zbuf, ssem)
        plsc.subcore_barrier()

        def group(g, carry):
            ibase = pl.multiple_of(wid * CPT + g * GRPC, 8)
            ir = pltpu.async_copy(ei_ref.at[0, pl.ds(ibase, GRPC)],
                                  idxr, isem)
            ic = pltpu.async_copy(ei_ref.at[1, pl.ds(ibase, GRPC)],
                                  idxc, isem)
            ir.wait()
            ic.wait()
            gds = [pltpu.async_copy(srcr.at[idxr.at[c]],
                                    gbuf.at[c % NBUF16], gsem)
                   for c in range(NBUF16 - 1)]
            scs = []
            for c in range(GRPC):
                if c >= 1:
                    scs[c - 1].wait()
                if c + NBUF16 - 1 < GRPC:
                    gds.append(pltpu.async_copy(
                        srcr.at[idxr.at[c + NBUF16 - 1]],
                        gbuf.at[(c + NBUF16 - 1) % NBUF16], gsem))
                gds[c].wait()
                scs.append(pltpu.async_copy(
                    gbuf.at[c % NBUF16], acc.at[idxc.at[c]], ssem,
                    add=True))
            scs[GRPC - 1].wait()
            return carry

        lax.fori_loop(0, NGRP, group, 0)
        plsc.subcore_barrier()

        @pl.when(cid == 0)
        def _():
            _writeback_blocks(sid, acc, out0, ssem)

        @pl.when(cid == 1)
        def _():
            _writeback_blocks(sid, acc, out1, ssem)

    return k(src, ei3)


_DOT = dict(preferred_element_type=f32, precision=lax.Precision.HIGHEST)


def _tc_w(c2i, c2w0, c2r0, lint, c2b):
    def body(i2_ref, w2_ref, r2_ref, lin_ref, b2_ref, coef_ref, scal_ref):
        lint = lin_ref[...]
        ca, cb = [], []
        cdsum = jnp.zeros((16, 1), f32)
        b1s = jnp.zeros((1, 1), f32)
        b2s = jnp.zeros((1, 1), f32)
        for kk in range(3):
            wt = jnp.dot(w2_ref[kk], lint, **_DOT)
            ca.append(jnp.dot(i2_ref[kk], wt, **_DOT))
            cb.append(jnp.dot(r2_ref[kk], wt, **_DOT))
            cdsum = cdsum + jnp.dot(r2_ref[kk], lint, **_DOT)
            b1s = b1s + jnp.dot(b2_ref[kk:kk + 1, :], wt, **_DOT)
            b2s = b2s + jnp.dot(b2_ref[kk:kk + 1, :], lint, **_DOT)
        coef_ref[...] = jnp.concatenate(
            ca + cb + [cdsum, jnp.zeros((16, 1), f32)], axis=1)
        scal_ref[...] = jnp.concatenate(
            [b1s, b2s, jnp.zeros((1, 6), f32)], axis=1)

    return pl.pallas_call(
        body,
        out_shape=[SDS((16, 8), f32), SDS((1, 8), f32)],
    )(c2i, c2w0, c2r0, lint, c2b)


def _tc_a(x, d0, d1, i1s, r1s, b1v):
    def body(x_ref, d0_ref, d1_ref, w_ref, wr_ref, b_ref,
             xs0_ref, xs1_ref, rpd_ref):
        deg = d0_ref[:, 0:1] + d1_ref[:, 0:1]
        dis = jnp.where(deg > 0.0, lax.rsqrt(jnp.maximum(deg, 1e-30)), 0.0)
        xs = jnp.dot(x_ref[...], w_ref[...], **_DOT) * dis
        pad = jnp.zeros((NB, 8), f32)
        xs0_ref[...] = jnp.concatenate([xs[:, :24], pad], axis=1)
        xs1_ref[...] = jnp.concatenate([xs[:, 24:], pad], axis=1)
        rp = jnp.dot(x_ref[...], wr_ref[...], **_DOT) + b_ref[...]
        rpd_ref[...] = jnp.concatenate(
            [rp, dis, jnp.zeros((NB, 15), f32)], axis=1)

    return pl.pallas_call(
        body,
        grid=(NSTEPS,),
        in_specs=[
            pl.BlockSpec((NB, 75), lambda i: (i, 0)),
            pl.BlockSpec((NB, 16), lambda i: (i, 0)),
            pl.BlockSpec((NB, 16), lambda i: (i, 0)),
            pl.BlockSpec((75, 48), lambda i: (0, 0)),
            pl.BlockSpec((75, 48), lambda i: (0, 0)),
            pl.BlockSpec((1, 48), lambda i: (0, 0)),
        ],
        out_specs=[
            pl.BlockSpec((NB, 32), lambda i: (i, 0)),
            pl.BlockSpec((NB, 32), lambda i: (i, 0)),
            pl.BlockSpec((NB, 64), lambda i: (i, 0)),
        ],
        out_shape=[SDS((N, 32), f32), SDS((N, 32), f32), SDS((N, 64), f32)],
    )(x, d0, d1, i1s, r1s, b1v)


def _tc_b(p0, p1, rpd, w1bd):
    def body(p0_ref, p1_ref, rpd_ref, w_ref, y0_ref, y1_ref):
        dis = rpd_ref[:, 48:49]
        p = jnp.concatenate([p0_ref[:, :24], p1_ref[:, :24]], axis=1)
        o1 = jnp.maximum(p * dis + rpd_ref[:, :48], 0.0)
        y = jnp.dot(o1, w_ref[...], **_DOT) * dis
        pad = jnp.zeros((NB, 8), f32)
        y0_ref[...] = jnp.concatenate([y[:, :24], pad], axis=1)
        y1_ref[...] = jnp.concatenate([y[:, 24:], pad], axis=1)

    return pl.pallas_call(
        body,
        grid=(NSTEPS,),
        in_specs=[
            pl.BlockSpec((NB, 32), lambda i: (i, 0)),
            pl.BlockSpec((NB, 32), lambda i: (i, 0)),
            pl.BlockSpec((NB, 64), lambda i: (i, 0)),
            pl.BlockSpec((48, 48), lambda i: (0, 0)),
        ],
        out_specs=[
            pl.BlockSpec((NB, 32), lambda i: (i, 0)),
            pl.BlockSpec((NB, 32), lambda i: (i, 0)),
        ],
        out_shape=[SDS((N, 32), f32), SDS((N, 32), f32)],
    )(p0, p1, rpd, w1bd)


def _tc_c(p0, p1, rpd, coef):
    def body(p0_ref, p1_ref, rpd_ref, coef_ref, t_ref):
        dis = rpd_ref[:, 48:49]
        p = jnp.concatenate([p0_ref[:, :24], p1_ref[:, :24]], axis=1)
        o2 = jnp.maximum(p * dis + rpd_ref[:, :48], 0.0)
        h1 = (o2[:, :16] + o2[:, 16:32] + o2[:, 32:]) * (1.0 / 3.0)
        ab = jnp.dot(h1, coef_ref[:, 0:6], **_DOT)
        dsum = jnp.dot(h1, coef_ref[:, 6:7], **_DOT)
        pad = jnp.zeros((NB, 8), f32)
        t_ref[...] = jnp.concatenate(
            [ab * dis, dis, dsum, pad], axis=1)

    return pl.pallas_call(
        body,
        grid=(NSTEPS,),
        in_specs=[
            pl.BlockSpec((NB, 32), lambda i: (i, 0)),
            pl.BlockSpec((NB, 32), lambda i: (i, 0)),
            pl.BlockSpec((NB, 64), lambda i: (i, 0)),
            pl.BlockSpec((16, 8), lambda i: (0, 0)),
        ],
        out_specs=pl.BlockSpec((NB, 16), lambda i: (i, 0)),
        out_shape=SDS((N, 16), f32),
    )(p0, p1, rpd, coef)


def _tc_d(q0, q1, t, scal):
    def body(q0_ref, q1_ref, t_ref, scal_ref, u_ref):
        acc3 = q0_ref[...] + q1_ref[...]
        dis = t_ref[:, 6:7]
        b1s = scal_ref[:, 0:1]
        src2 = acc3[:, 0:3] * dis * dis
        spart = (acc3[:, 3:4] + acc3[:, 4:5] + acc3[:, 5:6]
                 + b1s * acc3[:, 6:7]) * dis + t_ref[:, 7:8]
        pad = jnp.zeros((NB, 11), f32)
        u_ref[...] = jnp.concatenate([src2, spart, dis, pad], axis=1)

    return pl.pallas_call(
        body,
        grid=(NSTEPS,),
        in_specs=[
            pl.BlockSpec((NB, 16), lambda i: (i, 0)),
            pl.BlockSpec((NB, 16), lambda i: (i, 0)),
            pl.BlockSpec((NB, 16), lambda i: (i, 0)),
            pl.BlockSpec((1, 8), lambda i: (0, 0)),
        ],
        out_specs=pl.BlockSpec((NB, 16), lambda i: (i, 0)),
        out_shape=SDS((N, 16), f32),
    )(q0, q1, t, scal)


def _tc_e(r0, r1, u, batch2d, scal, linb):
    def body(r0_ref, r1_ref, u_ref, bt_ref, scal_ref, lb_ref,
             z_ref, zacc, nacc):
        i = pl.program_id(0)
        acc4 = r0_ref[...] + r1_ref[...]
        dis = u_ref[:, 4:5]
        s = (acc4[:, 0:1] + acc4[:, 1:2] + acc4[:, 2:3]) * dis \
            + u_ref[:, 3:4]
        iota = lax.broadcasted_iota(jnp.int32, (1, G), 1)
        m = (bt_ref[...] == iota).astype(f32)
        zp = jnp.sum(m * s, axis=0, keepdims=True)
        npp = jnp.sum(m, axis=0, keepdims=True)

        @pl.when(i == 0)
        def _():
            zacc[...] = jnp.zeros((1, G), f32)
            nacc[...] = jnp.zeros((1, G), f32)

        zacc[...] += zp
        nacc[...] += npp

        @pl.when(i == NSTEPS - 1)
        def _():
            b2s = scal_ref[:, 1:2]
            z_ref[...] = (zacc[...] + b2s * nacc[...]) * (1.0 / 3.0) \
                + lb_ref[...]

    return pl.pallas_call(
        body,
        grid=(NSTEPS,),
        in_specs=[
            pl.BlockSpec((NB, 16), lambda i: (i, 0)),
            pl.BlockSpec((NB, 16), lambda i: (i, 0)),
            pl.BlockSpec((NB, 16), lambda i: (i, 0)),
            pl.BlockSpec((NB, 1), lambda i: (i, 0)),
            pl.BlockSpec((1, 8), lambda i: (0, 0)),
            pl.BlockSpec((1, 1), lambda i: (0, 0)),
        ],
        out_specs=pl.BlockSpec((1, G), lambda i: (0, 0)),
        out_shape=SDS((1, G), f32),
        scratch_shapes=[pltpu.VMEM((1, G), f32), pltpu.VMEM((1, G), f32)],
    )(r0, r1, u, batch2d, scal, linb)


def kernel(x, edge_index, batch, c1_init, c1_w, c1_root, c1_bias,
           c2_init, c2_w, c2_root, c2_bias, lin_w, lin_b):
    ei3 = edge_index.reshape(2, NCH, CHUNK)
    batch2d = batch.reshape(N, 1)

    i1s = c1_init.transpose(1, 0, 2).reshape(75, 48)
    r1s = c1_root[0].transpose(1, 0, 2).reshape(75, 48)
    b1v = c1_bias[0].reshape(1, 48)
    w1bd = jnp.zeros((48, 48), f32)
    for kk in range(3):
        w1bd = w1bd.at[kk * 16:(kk + 1) * 16, kk * 16:(kk + 1) * 16].set(
            c1_w[0, kk])
    c2w0 = c2_w[0]
    c2r0 = c2_root[0]
    c2b = c2_bias[0].reshape(3, 64)
    lint = lin_w.reshape(64, 1)
    linb = lin_b.reshape(1, 1)

    coef, scal = _tc_w(c2_init, c2w0, c2r0, lint, c2b)
    d0, d1 = _sc_degree(ei3)
    xs0, xs1, rpd = _tc_a(x, d0, d1, i1s, r1s, b1v)
    p10, p11 = _sc_prop2x32(xs0, xs1, ei3)
    ys0, ys1 = _tc_b(p10, p11, rpd, w1bd)
    p20, p21 = _sc_prop2x32(ys0, ys1, ei3)
    t = _tc_c(p20, p21, rpd, coef)
    q0, q1 = _sc_prop16(t, ei3)
    u = _tc_d(q0, q1, t, scal)
    r0, r1 = _sc_prop16(u, ei3)
    zrow = _tc_e(r0, r1, u, batch2d, scal, linb)
    return zrow.reshape(G, 1)
